# baseline (speedup 1.0000x reference)
import functools

import jax
import jax.numpy as jnp
from jax import lax
from jax.experimental import pallas as pl
from jax.experimental.pallas import tpu as pltpu
from jax.experimental.pallas import tpu_sc as plsc

N_NODES = 10000
N_EDGES = 160000
D = 256

ROWS_BIG = 320
ROWS_SMALL = 312
TRASH = 320
ACC_ROWS = 321

MACRO = 1280
NUM_MACRO = N_EDGES // MACRO
GROUPS = MACRO // 16

CAP = 80
DRAIN_AT = 64


def _sc_agg_body(x_hbm, src_hbm, dst_hbm, agg_hbm,
                 acc, sbuf, dbuf, csrc, cloc, gidx, rows,
                 cur_smem, pend_smem, pcnt_smem, sem, isem):
    c = lax.axis_index("c")
    s = lax.axis_index("s")
    t = s * 2 + c
    lo = jnp.where(t < 2, t * ROWS_BIG,
                   2 * ROWS_BIG + (t - 2) * ROWS_SMALL)
    nrows = jnp.where(t < 2, ROWS_BIG, ROWS_SMALL)
    lo_v = jnp.full((16,), lo, jnp.int32)
    hi_v = jnp.full((16,), lo + nrows, jnp.int32)
    one_v = jnp.ones((16,), jnp.int32)
    zero_v = jnp.zeros((16,), jnp.int32)
    trash_v = jnp.full((16,), TRASH, jnp.int32)

    def zero_row(i, carry):
        for k in range(D // 16):
            acc[i, pl.ds(k * 16, 16)] = jnp.zeros((16,), jnp.float32)
        return carry

    lax.fori_loop(0, ACC_ROWS, zero_row, 0)
    cur_smem[0] = 0
    pend_smem[0] = 0
    pcnt_smem[0] = 0

    def issue_gather(slot, cnt):
        cnt_v = jnp.full((16,), cnt, jnp.int32)
        for jc in range(CAP // 16):
            eids = jnp.arange(jc * 16, jc * 16 + 16, dtype=jnp.int32)
            sv = csrc[pl.ds(slot * CAP + jc * 16, 16)]
            gidx[pl.ds(slot * CAP + jc * 16, 16)] = (
                jnp.where(eids < cnt_v, sv, eids))
        pltpu.async_copy(x_hbm.at[gidx.at[pl.ds(slot * CAP, CAP)]],
                         rows.at[pl.ds(slot * CAP, CAP)], sem)

    def wait_gather():
        pltpu.make_async_copy(x_hbm.at[gidx.at[pl.ds(0, CAP)]],
                              rows.at[pl.ds(0, CAP)], sem).wait()

    def accumulate(slot, n):
        n_v = jnp.full((16,), n, jnp.int32)

        def chunk_body(jc, carry):
            eids = jc * 16 + jnp.arange(16, dtype=jnp.int32)
            lvec = jnp.where(eids < n_v,
                             cloc[pl.ds(slot * CAP + jc * 16, 16)] - lo_v,
                             trash_v)
            for l in range(16):
                row = lvec[l]
                e = slot * CAP + jc * 16 + l
                vals = [rows[e, pl.ds(k * 16, 16)] for k in range(D // 16)]
                for k in range(D // 16):
                    plsc.addupdate(acc.at[row, pl.ds(k * 16, 16)], vals[k])
            return carry

        lax.fori_loop(0, CAP // 16, chunk_body, 0)

    def drain_step(newcnt):
        cur = cur_smem[0]
        issue_gather(cur, newcnt)

        @pl.when(pend_smem[0] == 1)
        def _():
            wait_gather()
            accumulate(1 - cur, pcnt_smem[0])

        pend_smem[0] = 1
        pcnt_smem[0] = newcnt
        cur_smem[0] = 1 - cur
        return 0

    pltpu.async_copy(src_hbm.at[pl.ds(0, MACRO)], sbuf.at[pl.ds(0, MACRO)],
                     isem)
    pltpu.async_copy(dst_hbm.at[pl.ds(0, MACRO)], dbuf.at[pl.ds(0, MACRO)],
                     isem)

    def wait_idx():
        pltpu.make_async_copy(src_hbm.at[pl.ds(0, MACRO)],
                              sbuf.at[pl.ds(0, MACRO)], isem).wait()
        pltpu.make_async_copy(dst_hbm.at[pl.ds(0, MACRO)],
                              dbuf.at[pl.ds(0, MACRO)], isem).wait()

    def macro_body(m, cnt_in):
        mslot = m % 2
        mbase = mslot * MACRO
        wait_idx()

        @pl.when(m < NUM_MACRO - 1)
        def _():
            noff = (m + 1) * MACRO
            nbase = ((m + 1) % 2) * MACRO
            pltpu.async_copy(src_hbm.at[pl.ds(noff, MACRO)],
                             sbuf.at[pl.ds(nbase, MACRO)], isem)
            pltpu.async_copy(dst_hbm.at[pl.ds(noff, MACRO)],
                             dbuf.at[pl.ds(nbase, MACRO)], isem)

        def group_body(g4, cnt_c):
            ds_, masks, svs, pcs = [], [], [], []
            for q in range(4):
                gb = mbase + (g4 * 4 + q) * 16
                d = dbuf[pl.ds(gb, 16)]
                mask = (d >= lo_v) & (d < hi_v)
                ds_.append(d)
                masks.append(mask)
                svs.append(sbuf[pl.ds(gb, 16)])
                pcs.append(jnp.sum(jnp.where(mask, one_v, zero_v)))
            for q in range(4):
                def append_q(cnt_now, q=q):
                    cur = cur_smem[0]
                    base_i = cur * CAP + cnt_now
                    plsc.store_compressed(cloc.at[pl.ds(base_i, 16)],
                                          ds_[q], mask=masks[q])
                    plsc.store_compressed(csrc.at[pl.ds(base_i, 16)],
                                          svs[q], mask=masks[q])
                    newcnt = cnt_now + pcs[q]
                    return lax.cond(newcnt >= DRAIN_AT, drain_step,
                                    lambda n: n, newcnt)
                cnt_c = lax.cond(pcs[q] > 0, append_q,
                                 lambda n: n, cnt_c)
            return cnt_c

        return lax.fori_loop(0, GROUPS // 4, group_body, cnt_in)

    cnt_fin = lax.fori_loop(0, NUM_MACRO, macro_body, 0)

    @pl.when(pend_smem[0] == 1)
    def _():
        wait_gather()
        accumulate(1 - cur_smem[0], pcnt_smem[0])

    @pl.when(cnt_fin > 0)
    def _():
        cur = cur_smem[0]
        issue_gather(cur, cnt_fin)
        wait_gather()
        accumulate(cur, cnt_fin)

    @pl.when(t < 2)
    def _():
        pltpu.sync_copy(acc.at[pl.ds(0, ROWS_BIG)],
                        agg_hbm.at[pl.ds(t * ROWS_BIG, ROWS_BIG)])

    @pl.when(t >= 2)
    def _():
        st = 2 * ROWS_BIG + (t - 2) * ROWS_SMALL
        pltpu.sync_copy(acc.at[pl.ds(0, ROWS_SMALL)],
                        agg_hbm.at[pl.ds(st, ROWS_SMALL)])


_sc_aggregate = functools.partial(
    pl.kernel,
    out_type=jax.ShapeDtypeStruct((N_NODES, D), jnp.float32),
    mesh=plsc.VectorSubcoreMesh(core_axis_name="c", subcore_axis_name="s"),
    compiler_params=pltpu.CompilerParams(needs_layout_passes=False),
    scratch_types=[
        pltpu.VMEM((ACC_ROWS, D), jnp.float32),
        pltpu.VMEM((2 * MACRO,), jnp.int32),
        pltpu.VMEM((2 * MACRO,), jnp.int32),
        pltpu.VMEM((2 * CAP,), jnp.int32),
        pltpu.VMEM((2 * CAP,), jnp.int32),
        pltpu.VMEM((2 * CAP,), jnp.int32),
        pltpu.VMEM((2 * CAP, D), jnp.float32),
        pltpu.SMEM((1,), jnp.int32),
        pltpu.SMEM((1,), jnp.int32),
        pltpu.SMEM((1,), jnp.int32),
        pltpu.SemaphoreType.DMA,
        pltpu.SemaphoreType.DMA,
    ],
)(_sc_agg_body)


def _mm_body(agg_ref, w_ref, b_ref, out_ref):
    out_ref[...] = lax.dot_general(
        agg_ref[...], w_ref[...],
        (((1,), (1,)), ((), ())),
        preferred_element_type=jnp.float32,
    ) + b_ref[...]


_ROWS_BLK = 2000


def _tc_linear(agg, W, b2d):
    return pl.pallas_call(
        _mm_body,
        grid=(N_NODES // _ROWS_BLK,),
        in_specs=[
            pl.BlockSpec((_ROWS_BLK, D), lambda i: (i, 0)),
            pl.BlockSpec((D, D), lambda i: (0, 0)),
            pl.BlockSpec((1, D), lambda i: (0, 0)),
        ],
        out_specs=pl.BlockSpec((_ROWS_BLK, D), lambda i: (i, 0)),
        out_shape=jax.ShapeDtypeStruct((N_NODES, D), jnp.float32),
    )(agg, W, b2d)


def kernel(x, edge_index, W, b):
    src = edge_index[0]
    dst = edge_index[1]
    agg = _sc_aggregate(x, src, dst)
    return _tc_linear(agg, W, b.reshape(1, D))


# speedup vs baseline: 1.0681x; 1.0681x over previous
import functools

import jax
import jax.numpy as jnp
from jax import lax
from jax.experimental import pallas as pl
from jax.experimental.pallas import tpu as pltpu
from jax.experimental.pallas import tpu_sc as plsc

N_NODES = 10000
N_EDGES = 160000
D = 256

ROWS_BIG = 320
ROWS_SMALL = 312
TRASH = 320
ACC_ROWS = 321

MACRO = 1280
NUM_MACRO = N_EDGES // MACRO
GROUPS = MACRO // 16

CAP = 80
DRAIN_AT = 64


def _sc_agg_body(x_hbm, src_hbm, dst_hbm, agg_hbm,
                 acc, sbuf, dbuf, csrc, cloc, gidx, rows,
                 cur_smem, pend_smem, pcnt_smem, sem, isem):
    c = lax.axis_index("c")
    s = lax.axis_index("s")
    t = s * 2 + c
    lo = jnp.where(t < 2, t * ROWS_BIG,
                   2 * ROWS_BIG + (t - 2) * ROWS_SMALL)
    nrows = jnp.where(t < 2, ROWS_BIG, ROWS_SMALL)
    lo_v = jnp.full((16,), lo, jnp.int32)
    hi_v = jnp.full((16,), lo + nrows, jnp.int32)
    one_v = jnp.ones((16,), jnp.int32)
    zero_v = jnp.zeros((16,), jnp.int32)
    trash_v = jnp.full((16,), TRASH, jnp.int32)

    def zero_row(i, carry):
        for k in range(D // 16):
            acc[i, pl.ds(k * 16, 16)] = jnp.zeros((16,), jnp.float32)
        return carry

    lax.fori_loop(0, ACC_ROWS, zero_row, 0)
    cur_smem[0] = 0
    pend_smem[0] = 0
    pcnt_smem[0] = 0

    def issue_gather(slot, cnt):
        cnt_v = jnp.full((16,), cnt, jnp.int32)
        for jc in range(CAP // 16):
            eids = jnp.arange(jc * 16, jc * 16 + 16, dtype=jnp.int32)
            sv = csrc[pl.ds(slot * CAP + jc * 16, 16)]
            gidx[pl.ds(slot * CAP + jc * 16, 16)] = (
                jnp.where(eids < cnt_v, sv, eids))
        pltpu.async_copy(x_hbm.at[gidx.at[pl.ds(slot * CAP, CAP)]],
                         rows.at[pl.ds(slot * CAP, CAP)], sem)

    def wait_gather():
        pltpu.make_async_copy(x_hbm.at[gidx.at[pl.ds(0, CAP)]],
                              rows.at[pl.ds(0, CAP)], sem).wait()

    def accumulate(slot, n):
        n_v = jnp.full((16,), n, jnp.int32)

        def chunk_body(jc, carry):
            eids = jc * 16 + jnp.arange(16, dtype=jnp.int32)
            lvec = jnp.where(eids < n_v,
                             cloc[pl.ds(slot * CAP + jc * 16, 16)] - lo_v,
                             trash_v)
            for l in range(16):
                row = lvec[l]
                e = slot * CAP + jc * 16 + l
                vals = [rows[e, pl.ds(k * 16, 16)] for k in range(D // 16)]
                for k in range(D // 16):
                    plsc.addupdate(acc.at[row, pl.ds(k * 16, 16)], vals[k])
            return carry

        lax.fori_loop(0, CAP // 16, chunk_body, 0)

    def drain_step(newcnt):
        cur = cur_smem[0]
        issue_gather(cur, newcnt)

        @pl.when(pend_smem[0] == 1)
        def _():
            wait_gather()
            accumulate(1 - cur, pcnt_smem[0])

        pend_smem[0] = 1
        pcnt_smem[0] = newcnt
        cur_smem[0] = 1 - cur
        return 0

    pltpu.async_copy(src_hbm.at[pl.ds(0, MACRO)], sbuf.at[pl.ds(0, MACRO)],
                     isem)
    pltpu.async_copy(dst_hbm.at[pl.ds(0, MACRO)], dbuf.at[pl.ds(0, MACRO)],
                     isem)

    def wait_idx():
        pltpu.make_async_copy(src_hbm.at[pl.ds(0, MACRO)],
                              sbuf.at[pl.ds(0, MACRO)], isem).wait()
        pltpu.make_async_copy(dst_hbm.at[pl.ds(0, MACRO)],
                              dbuf.at[pl.ds(0, MACRO)], isem).wait()

    def macro_body(m, cnt_in):
        mslot = m % 2
        mbase = mslot * MACRO
        wait_idx()

        @pl.when(m < NUM_MACRO - 1)
        def _():
            noff = (m + 1) * MACRO
            nbase = ((m + 1) % 2) * MACRO
            pltpu.async_copy(src_hbm.at[pl.ds(noff, MACRO)],
                             sbuf.at[pl.ds(nbase, MACRO)], isem)
            pltpu.async_copy(dst_hbm.at[pl.ds(noff, MACRO)],
                             dbuf.at[pl.ds(nbase, MACRO)], isem)

        def group_body(g4, cnt_c):
            ds_, masks, svs, pcs = [], [], [], []
            for q in range(4):
                gb = mbase + (g4 * 4 + q) * 16
                d = dbuf[pl.ds(gb, 16)]
                mask = (d >= lo_v) & (d < hi_v)
                ds_.append(d)
                masks.append(mask)
                svs.append(sbuf[pl.ds(gb, 16)])
                pcs.append(jnp.sum(jnp.where(mask, one_v, zero_v)))
            for q in range(4):
                cur = cur_smem[0]
                base_i = cur * CAP + cnt_c
                plsc.store_compressed(cloc.at[pl.ds(base_i, 16)],
                                      ds_[q], mask=masks[q])
                plsc.store_compressed(csrc.at[pl.ds(base_i, 16)],
                                      svs[q], mask=masks[q])
                newcnt = cnt_c + pcs[q]
                cnt_c = lax.cond(newcnt >= DRAIN_AT, drain_step,
                                 lambda n: n, newcnt)
            return cnt_c

        return lax.fori_loop(0, GROUPS // 4, group_body, cnt_in)

    cnt_fin = lax.fori_loop(0, NUM_MACRO, macro_body, 0)

    @pl.when(pend_smem[0] == 1)
    def _():
        wait_gather()
        accumulate(1 - cur_smem[0], pcnt_smem[0])

    @pl.when(cnt_fin > 0)
    def _():
        cur = cur_smem[0]
        issue_gather(cur, cnt_fin)
        wait_gather()
        accumulate(cur, cnt_fin)

    @pl.when(t < 2)
    def _():
        pltpu.sync_copy(acc.at[pl.ds(0, ROWS_BIG)],
                        agg_hbm.at[pl.ds(t * ROWS_BIG, ROWS_BIG)])

    @pl.when(t >= 2)
    def _():
        st = 2 * ROWS_BIG + (t - 2) * ROWS_SMALL
        pltpu.sync_copy(acc.at[pl.ds(0, ROWS_SMALL)],
                        agg_hbm.at[pl.ds(st, ROWS_SMALL)])


_sc_aggregate = functools.partial(
    pl.kernel,
    out_type=jax.ShapeDtypeStruct((N_NODES, D), jnp.float32),
    mesh=plsc.VectorSubcoreMesh(core_axis_name="c", subcore_axis_name="s"),
    compiler_params=pltpu.CompilerParams(needs_layout_passes=False),
    scratch_types=[
        pltpu.VMEM((ACC_ROWS, D), jnp.float32),
        pltpu.VMEM((2 * MACRO,), jnp.int32),
        pltpu.VMEM((2 * MACRO,), jnp.int32),
        pltpu.VMEM((2 * CAP,), jnp.int32),
        pltpu.VMEM((2 * CAP,), jnp.int32),
        pltpu.VMEM((2 * CAP,), jnp.int32),
        pltpu.VMEM((2 * CAP, D), jnp.float32),
        pltpu.SMEM((1,), jnp.int32),
        pltpu.SMEM((1,), jnp.int32),
        pltpu.SMEM((1,), jnp.int32),
        pltpu.SemaphoreType.DMA,
        pltpu.SemaphoreType.DMA,
    ],
)(_sc_agg_body)


def _mm_body(agg_ref, w_ref, b_ref, out_ref):
    out_ref[...] = lax.dot_general(
        agg_ref[...], w_ref[...],
        (((1,), (1,)), ((), ())),
        preferred_element_type=jnp.float32,
    ) + b_ref[...]


_ROWS_BLK = 2000


def _tc_linear(agg, W, b2d):
    return pl.pallas_call(
        _mm_body,
        grid=(N_NODES // _ROWS_BLK,),
        in_specs=[
            pl.BlockSpec((_ROWS_BLK, D), lambda i: (i, 0)),
            pl.BlockSpec((D, D), lambda i: (0, 0)),
            pl.BlockSpec((1, D), lambda i: (0, 0)),
        ],
        out_specs=pl.BlockSpec((_ROWS_BLK, D), lambda i: (i, 0)),
        out_shape=jax.ShapeDtypeStruct((N_NODES, D), jnp.float32),
    )(agg, W, b2d)


def kernel(x, edge_index, W, b):
    src = edge_index[0]
    dst = edge_index[1]
    agg = _sc_aggregate(x, src, dst)
    return _tc_linear(agg, W, b.reshape(1, D))
